# baseline (speedup 1.0000x reference)
import functools
import math

import jax
import jax.numpy as jnp
from jax.experimental import pallas as pl
from jax.experimental.pallas import tpu as pltpu

_LN_EPS = 1e-12
_INV_SQRT2 = 0.7071067811865476
_NUM_HEADS = 8
_NCORES = 1


def _ln(x, g, b):
    mu = jnp.mean(x, axis=-1, keepdims=True)
    ex2 = jnp.mean(x * x, axis=-1, keepdims=True)
    var = jnp.maximum(ex2 - mu * mu, 0.0)
    return (x - mu) * jax.lax.rsqrt(var + _LN_EPS) * g + b


def _gelu(x):
    return 0.5 * x * (1.0 + jax.lax.erf(x * _INV_SQRT2))


def _fused_kernel(S,
                  ids_ref,
                  wemb_hbm,
                  posx_ref, typ_ref, embg_ref, embb_ref,
                  mask_ref,
                  wqkv_ref, bqkv_ref, wo_ref, bo_ref,
                  l1g_ref, l1b_ref, w1_ref, b1_ref, w2_ref, b2_ref,
                  l2g_ref, l2b_ref,
                  o_ref,
                  qkv_scr, head_scr, ctx_scr, sems):
    c = pl.program_id(0)
    l = pl.program_id(1)
    n_layers = pl.num_programs(1)

    T, H = o_ref.shape[1], o_ref.shape[2]
    nh = _NUM_HEADS
    hd = H // nh
    R = T // S
    nb = R * nh
    scale = 1.0 / math.sqrt(hd)
    bf16, f32 = jnp.bfloat16, jnp.float32

    NSEM = 16
    BANK = T // NSEM

    @pl.when(l == 0)
    def _():
        def issue(i, carry):
            for j in range(8):
                tid = ids_ref[c, i * 8 + j]
                pltpu.make_async_copy(
                    wemb_hbm.at[pl.ds(tid, 1)],
                    o_ref.at[0, pl.ds(i * 8 + j, 1)],
                    sems.at[jnp.bitwise_and(i * 8 + j, NSEM - 1)]).start()
            return carry
        jax.lax.fori_loop(0, T // 8, issue, 0)

        for s in range(NSEM):
            pltpu.make_async_copy(
                wemb_hbm.at[pl.ds(0, BANK)],
                o_ref.at[0, pl.ds(0, BANK)],
                sems.at[s]).wait()

        S_, H_ = posx_ref.shape
        posT = posx_ref[...] + typ_ref[...]
        emb = o_ref[0].reshape(T // S_, S_, H_) + posT[None]
        o_ref[0] = _ln(emb, embg_ref[...], embb_ref[...]).reshape(T, H)

    @pl.when(l == 0)
    def _():
        head_scr[...] = jnp.zeros_like(head_scr)

    x = o_ref[0]

    qkv_scr[...] = jnp.dot(x.astype(bf16), wqkv_ref[0],
                           preferred_element_type=f32) + bqkv_ref[0]
    for r in range(R):
        rs = pl.ds(r * S, S)
        for h in range(nh):
            bidx = r * nh + h
            head_scr[0, bidx, :, :hd] = (qkv_scr[rs, pl.ds(h * hd, hd)]
                                         * scale).astype(bf16)
            head_scr[1, bidx, :, :hd] = qkv_scr[rs, pl.ds(H + h * hd, hd)].astype(bf16)
            head_scr[2, bidx, :, :hd] = qkv_scr[rs, pl.ds(2 * H + h * hd, hd)].astype(bf16)

    sc = jnp.einsum("bqd,bkd->bqk", head_scr[0], head_scr[1],
                    preferred_element_type=f32)
    msk = mask_ref[0]
    sc = sc + jnp.broadcast_to(msk[:, None], (R, nh, 1, S)).reshape(nb, 1, S)
    p = jnp.exp(sc)
    p = p * pl.reciprocal(jnp.sum(p, axis=-1, keepdims=True), approx=True)
    ctx = jnp.einsum("bqk,bkd->bqd", p.astype(bf16), head_scr[2],
                     preferred_element_type=f32)

    for r in range(R):
        rs = pl.ds(r * S, S)
        for h in range(nh):
            ctx_scr[rs, pl.ds(h * hd, hd)] = ctx[r * nh + h, :, :hd].astype(bf16)

    attn = jnp.dot(ctx_scr[...], wo_ref[0],
                   preferred_element_type=f32) + bo_ref[0]
    h1 = _ln(x + attn, l1g_ref[0], l1b_ref[0])

    ff = jnp.dot(h1.astype(bf16), w1_ref[0],
                 preferred_element_type=f32) + b1_ref[0]
    ff = _gelu(ff)
    ff2 = jnp.dot(ff.astype(bf16), w2_ref[0],
                  preferred_element_type=f32) + b2_ref[0]

    o_ref[0] = _ln(h1 + ff2, l2g_ref[0], l2b_ref[0])


def kernel(word_emb, pos_emb, type_emb, emb_ln_g, emb_ln_b,
           wqkv, bqkv, wo, bo, ln1_g, ln1_b, w1, b1, w2, b2,
           ln2_g, ln2_b, input_ids, attention_mask):
    B, S = input_ids.shape
    H = word_emb.shape[1]
    L = wqkv.shape[0]
    I = w1.shape[2]
    nh = _NUM_HEADS
    hd = H // nh
    R = B // _NCORES
    T = R * S

    if attention_mask is None:
        attention_mask = jnp.ones((B, S), jnp.float32)
    mask_bias = ((1.0 - attention_mask.astype(jnp.float32))
                 * jnp.finfo(jnp.float32).min).reshape(_NCORES, R, 1, S)
    ids2 = input_ids.astype(jnp.int32).reshape(_NCORES, T)

    def cmap(shape):
        return pl.BlockSpec(shape, lambda c, l, ids: (0,) * len(shape))

    def wspec(d1, d2):
        return pl.BlockSpec((1, d1, d2), lambda c, l, ids: (l, 0, 0))

    grid_spec = pltpu.PrefetchScalarGridSpec(
        num_scalar_prefetch=1,
        grid=(_NCORES, L),
        in_specs=[
            pl.BlockSpec(memory_space=pl.ANY),
            cmap((S, H)),
            cmap((1, H)),
            cmap((1, H)), cmap((1, H)),
            pl.BlockSpec((1, R, 1, S), lambda c, l, ids: (c, 0, 0, 0)),
            wspec(H, 3 * H), wspec(1, 3 * H),
            wspec(H, H), wspec(1, H),
            wspec(1, H), wspec(1, H),
            wspec(H, I), wspec(1, I),
            wspec(I, H), wspec(1, H),
            wspec(1, H), wspec(1, H),
        ],
        out_specs=pl.BlockSpec((1, T, H), lambda c, l, ids: (c, 0, 0)),
        scratch_shapes=[
            pltpu.VMEM((T, 3 * H), jnp.float32),
            pltpu.VMEM((3, R * nh, S, 2 * hd), jnp.bfloat16),
            pltpu.VMEM((T, H), jnp.bfloat16),
            pltpu.SemaphoreType.DMA((16,)),
        ],
    )

    out = pl.pallas_call(
        functools.partial(_fused_kernel, S),
        out_shape=jax.ShapeDtypeStruct((_NCORES, T, H), jnp.float32),
        grid_spec=grid_spec,
        compiler_params=pltpu.CompilerParams(
            dimension_semantics=("core_parallel", "arbitrary"),
            vmem_limit_bytes=56 * 1024 * 1024),
    )(ids2, word_emb, pos_emb[:S], type_emb[0:1],
      emb_ln_g.reshape(1, H), emb_ln_b.reshape(1, H), mask_bias,
      wqkv, bqkv, wo, bo, ln1_g, ln1_b, w1, b1, w2, b2, ln2_g, ln2_b)

    return out.reshape(B, S, H)


# speedup vs baseline: 1.3835x; 1.3835x over previous
import functools
import math

import jax
import jax.numpy as jnp
from jax.experimental import pallas as pl
from jax.experimental.pallas import tpu as pltpu

_LN_EPS = 1e-12
_INV_SQRT2 = 0.7071067811865476
_NUM_HEADS = 8
_NCORES = 1


def _ln(x, g, b):
    mu = jnp.mean(x, axis=-1, keepdims=True)
    ex2 = jnp.mean(x * x, axis=-1, keepdims=True)
    var = jnp.maximum(ex2 - mu * mu, 0.0)
    return (x - mu) * jax.lax.rsqrt(var + _LN_EPS) * g + b


def _gelu(x):
    return 0.5 * x * (1.0 + jax.lax.erf(x * _INV_SQRT2))


def _fused_kernel(S,
                  ids_ref,
                  wemb_hbm,
                  posx_ref, typ_ref, embg_ref, embb_ref,
                  mask_ref,
                  wqkv_ref, bqkv_ref, wo_ref, bo_ref,
                  l1g_ref, l1b_ref, w1_ref, b1_ref, w2_ref, b2_ref,
                  l2g_ref, l2b_ref,
                  o_ref,
                  qkv_scr, head_scr, pair_scr, vbd_scr, ctx_scr, sems):
    c = pl.program_id(0)
    l = pl.program_id(1)
    n_layers = pl.num_programs(1)

    T, H = o_ref.shape[1], o_ref.shape[2]
    nh = _NUM_HEADS
    hd = H // nh
    R = T // S
    nb = R * nh
    scale = 1.0 / math.sqrt(hd)
    bf16, f32 = jnp.bfloat16, jnp.float32

    NSEM = 16
    BANK = T // NSEM

    @pl.when(l == 0)
    def _():
        def issue(i, carry):
            for j in range(8):
                tid = ids_ref[c, i * 8 + j]
                pltpu.make_async_copy(
                    wemb_hbm.at[pl.ds(tid, 1)],
                    o_ref.at[0, pl.ds(i * 8 + j, 1)],
                    sems.at[jnp.bitwise_and(i * 8 + j, NSEM - 1)]).start()
            return carry
        jax.lax.fori_loop(0, T // 8, issue, 0)

        for s in range(NSEM):
            pltpu.make_async_copy(
                wemb_hbm.at[pl.ds(0, BANK)],
                o_ref.at[0, pl.ds(0, BANK)],
                sems.at[s]).wait()

        S_, H_ = posx_ref.shape
        posT = posx_ref[...] + typ_ref[...]
        emb = o_ref[0].reshape(T // S_, S_, H_) + posT[None]
        o_ref[0] = _ln(emb, embg_ref[...], embb_ref[...]).reshape(T, H)

    @pl.when(l == 0)
    def _():
        vbd_scr[...] = jnp.zeros_like(vbd_scr)

    x = o_ref[0]

    qkv_scr[...] = jnp.dot(x.astype(bf16), wqkv_ref[0],
                           preferred_element_type=f32) + bqkv_ref[0]
    for r in range(R):
        rs = pl.ds(r * S, S)
        for h in range(nh):
            bidx = r * nh + h
            head_scr[0, bidx] = (qkv_scr[rs, pl.ds(h * hd, hd)]
                                 * scale).astype(bf16)
            head_scr[1, bidx] = qkv_scr[rs, pl.ds(H + h * hd, hd)].astype(bf16)
            vbd_scr[(r * nh + h) // 2,
                    pl.ds((h % 2) * S, S),
                    pl.ds((h % 2) * hd, hd)] = qkv_scr[rs, pl.ds(2 * H + h * hd, hd)].astype(bf16)

    sc = jnp.einsum("bqd,bkd->bqk", head_scr[0], head_scr[1],
                    preferred_element_type=f32)
    msk = mask_ref[0]
    sc = sc + jnp.broadcast_to(msk[:, None], (R, nh, 1, S)).reshape(nb, 1, S)
    p = jnp.exp(sc)
    rcp = pl.reciprocal(jnp.sum(p, axis=-1, keepdims=True), approx=True)
    for b2 in range(nb // 2):
        pair_scr[b2, :, pl.ds(0, S)] = (p[2 * b2] * rcp[2 * b2]).astype(bf16)
        pair_scr[b2, :, pl.ds(S, S)] = (p[2 * b2 + 1] * rcp[2 * b2 + 1]).astype(bf16)
    ctx = jnp.einsum("bqk,bkd->bqd", pair_scr[...], vbd_scr[...],
                     preferred_element_type=f32)

    for r in range(R):
        rs = pl.ds(r * S, S)
        for hp in range(nh // 2):
            ctx_scr[rs, pl.ds(hp * 2 * hd, 2 * hd)] = ctx[r * nh // 2 + hp].astype(bf16)

    attn = jnp.dot(ctx_scr[...], wo_ref[0],
                   preferred_element_type=f32) + bo_ref[0]
    h1 = _ln(x + attn, l1g_ref[0], l1b_ref[0])

    ff = jnp.dot(h1.astype(bf16), w1_ref[0],
                 preferred_element_type=f32) + b1_ref[0]
    ff = _gelu(ff)
    ff2 = jnp.dot(ff.astype(bf16), w2_ref[0],
                  preferred_element_type=f32) + b2_ref[0]

    o_ref[0] = _ln(h1 + ff2, l2g_ref[0], l2b_ref[0])


def kernel(word_emb, pos_emb, type_emb, emb_ln_g, emb_ln_b,
           wqkv, bqkv, wo, bo, ln1_g, ln1_b, w1, b1, w2, b2,
           ln2_g, ln2_b, input_ids, attention_mask):
    B, S = input_ids.shape
    H = word_emb.shape[1]
    L = wqkv.shape[0]
    I = w1.shape[2]
    nh = _NUM_HEADS
    hd = H // nh
    R = B // _NCORES
    T = R * S

    if attention_mask is None:
        attention_mask = jnp.ones((B, S), jnp.float32)
    mask_bias = ((1.0 - attention_mask.astype(jnp.float32))
                 * jnp.finfo(jnp.float32).min).reshape(_NCORES, R, 1, S)
    ids2 = input_ids.astype(jnp.int32).reshape(_NCORES, T)

    def cmap(shape):
        return pl.BlockSpec(shape, lambda c, l, ids: (0,) * len(shape))

    def wspec(d1, d2):
        return pl.BlockSpec((1, d1, d2), lambda c, l, ids: (l, 0, 0))

    grid_spec = pltpu.PrefetchScalarGridSpec(
        num_scalar_prefetch=1,
        grid=(_NCORES, L),
        in_specs=[
            pl.BlockSpec(memory_space=pl.ANY),
            cmap((S, H)),
            cmap((1, H)),
            cmap((1, H)), cmap((1, H)),
            pl.BlockSpec((1, R, 1, S), lambda c, l, ids: (c, 0, 0, 0)),
            wspec(H, 3 * H), wspec(1, 3 * H),
            wspec(H, H), wspec(1, H),
            wspec(1, H), wspec(1, H),
            wspec(H, I), wspec(1, I),
            wspec(I, H), wspec(1, H),
            wspec(1, H), wspec(1, H),
        ],
        out_specs=pl.BlockSpec((1, T, H), lambda c, l, ids: (c, 0, 0)),
        scratch_shapes=[
            pltpu.VMEM((T, 3 * H), jnp.float32),
            pltpu.VMEM((2, R * nh, S, hd), jnp.bfloat16),
            pltpu.VMEM((R * nh // 2, S, 2 * S), jnp.bfloat16),
            pltpu.VMEM((R * nh // 2, 2 * S, 2 * hd), jnp.bfloat16),
            pltpu.VMEM((T, H), jnp.bfloat16),
            pltpu.SemaphoreType.DMA((16,)),
        ],
    )

    out = pl.pallas_call(
        functools.partial(_fused_kernel, S),
        out_shape=jax.ShapeDtypeStruct((_NCORES, T, H), jnp.float32),
        grid_spec=grid_spec,
        compiler_params=pltpu.CompilerParams(
            dimension_semantics=("core_parallel", "arbitrary"),
            vmem_limit_bytes=56 * 1024 * 1024),
    )(ids2, word_emb, pos_emb[:S], type_emb[0:1],
      emb_ln_g.reshape(1, H), emb_ln_b.reshape(1, H), mask_bias,
      wqkv, bqkv, wo, bo, ln1_g, ln1_b, w1, b1, w2, b2, ln2_g, ln2_b)

    return out.reshape(B, S, H)


# speedup vs baseline: 1.4130x; 1.0213x over previous
import functools
import math

import jax
import jax.numpy as jnp
from jax.experimental import pallas as pl
from jax.experimental.pallas import tpu as pltpu

_LN_EPS = 1e-12
_INV_SQRT2 = 0.7071067811865476
_NUM_HEADS = 8
_NCORES = 1


def _ln(x, g, b):
    mu = jnp.mean(x, axis=-1, keepdims=True)
    ex2 = jnp.mean(x * x, axis=-1, keepdims=True)
    var = jnp.maximum(ex2 - mu * mu, 0.0)
    return (x - mu) * jax.lax.rsqrt(var + _LN_EPS) * g + b


def _gelu(x):
    return 0.5 * x * (1.0 + jax.lax.erf(x * _INV_SQRT2))


def _fused_kernel(S,
                  ids_ref,
                  wemb_hbm,
                  posx_ref, typ_ref, embg_ref, embb_ref,
                  mask_ref,
                  wqkv_ref, bqkv_ref, wo_ref, bo_ref,
                  l1g_ref, l1b_ref, w1_ref, b1_ref, w2_ref, b2_ref,
                  l2g_ref, l2b_ref,
                  o_ref,
                  qkv_scr, qpair_scr, kbd_scr, pair_scr, vbd_scr,
                  ctx_scr, sems):
    c = pl.program_id(0)
    l = pl.program_id(1)
    n_layers = pl.num_programs(1)

    T, H = o_ref.shape[1], o_ref.shape[2]
    nh = _NUM_HEADS
    hd = H // nh
    R = T // S
    nb = R * nh
    scale = 1.0 / math.sqrt(hd)
    bf16, f32 = jnp.bfloat16, jnp.float32

    NSEM = 16
    BANK = T // NSEM

    @pl.when(l == 0)
    def _():
        def issue(i, carry):
            for j in range(8):
                tid = ids_ref[c, i * 8 + j]
                pltpu.make_async_copy(
                    wemb_hbm.at[pl.ds(tid, 1)],
                    o_ref.at[0, pl.ds(i * 8 + j, 1)],
                    sems.at[jnp.bitwise_and(i * 8 + j, NSEM - 1)]).start()
            return carry
        jax.lax.fori_loop(0, T // 8, issue, 0)

        for s in range(NSEM):
            pltpu.make_async_copy(
                wemb_hbm.at[pl.ds(0, BANK)],
                o_ref.at[0, pl.ds(0, BANK)],
                sems.at[s]).wait()

        S_, H_ = posx_ref.shape
        posT = posx_ref[...] + typ_ref[...]
        emb = o_ref[0].reshape(T // S_, S_, H_) + posT[None]
        o_ref[0] = _ln(emb, embg_ref[...], embb_ref[...]).reshape(T, H)

    @pl.when(l == 0)
    def _():
        vbd_scr[...] = jnp.zeros_like(vbd_scr)
        kbd_scr[...] = jnp.zeros_like(kbd_scr)

    x = o_ref[0]

    qkv_scr[...] = jnp.dot(x.astype(bf16), wqkv_ref[0],
                           preferred_element_type=f32) + bqkv_ref[0]
    for r in range(R):
        rs = pl.ds(r * S, S)
        for h in range(nh):
            b2 = (r * nh + h) // 2
            sl = (h % 2) * hd
            qpair_scr[b2, :, pl.ds(sl, hd)] = (qkv_scr[rs, pl.ds(h * hd, hd)]
                                               * scale).astype(bf16)
            kbd_scr[b2, pl.ds((h % 2) * S, S),
                    pl.ds(sl, hd)] = qkv_scr[rs, pl.ds(H + h * hd, hd)].astype(bf16)
            vbd_scr[b2, pl.ds((h % 2) * S, S),
                    pl.ds(sl, hd)] = qkv_scr[rs, pl.ds(2 * H + h * hd, hd)].astype(bf16)

    sc = jnp.einsum("bqd,bkd->bqk", qpair_scr[...], kbd_scr[...],
                    preferred_element_type=f32)
    msk = mask_ref[0]
    m2 = jnp.concatenate([msk, msk], axis=-1)
    sc = sc + jnp.broadcast_to(m2[:, None], (R, nh // 2, 1, 2 * S)).reshape(
        nb // 2, 1, 2 * S)
    p = jnp.exp(sc)
    r0 = pl.reciprocal(jnp.sum(p[:, :, :S], axis=-1, keepdims=True), approx=True)
    r1 = pl.reciprocal(jnp.sum(p[:, :, S:], axis=-1, keepdims=True), approx=True)
    pair_scr[:, :, pl.ds(0, S)] = (p[:, :, :S] * r0).astype(bf16)
    pair_scr[:, :, pl.ds(S, S)] = (p[:, :, S:] * r1).astype(bf16)
    ctx = jnp.einsum("bqk,bkd->bqd", pair_scr[...], vbd_scr[...],
                     preferred_element_type=f32)

    for r in range(R):
        rs = pl.ds(r * S, S)
        for hp in range(nh // 2):
            ctx_scr[rs, pl.ds(hp * 2 * hd, 2 * hd)] = ctx[r * nh // 2 + hp].astype(bf16)

    attn = jnp.dot(ctx_scr[...], wo_ref[0],
                   preferred_element_type=f32) + bo_ref[0]
    h1 = _ln(x + attn, l1g_ref[0], l1b_ref[0])

    ff = jnp.dot(h1.astype(bf16), w1_ref[0],
                 preferred_element_type=f32) + b1_ref[0]
    ff = _gelu(ff)
    ff2 = jnp.dot(ff.astype(bf16), w2_ref[0],
                  preferred_element_type=f32) + b2_ref[0]

    o_ref[0] = _ln(h1 + ff2, l2g_ref[0], l2b_ref[0])


def kernel(word_emb, pos_emb, type_emb, emb_ln_g, emb_ln_b,
           wqkv, bqkv, wo, bo, ln1_g, ln1_b, w1, b1, w2, b2,
           ln2_g, ln2_b, input_ids, attention_mask):
    B, S = input_ids.shape
    H = word_emb.shape[1]
    L = wqkv.shape[0]
    I = w1.shape[2]
    nh = _NUM_HEADS
    hd = H // nh
    R = B // _NCORES
    T = R * S

    if attention_mask is None:
        attention_mask = jnp.ones((B, S), jnp.float32)
    mask_bias = ((1.0 - attention_mask.astype(jnp.float32))
                 * jnp.finfo(jnp.float32).min).reshape(_NCORES, R, 1, S)
    ids2 = input_ids.astype(jnp.int32).reshape(_NCORES, T)

    def cmap(shape):
        return pl.BlockSpec(shape, lambda c, l, ids: (0,) * len(shape))

    def wspec(d1, d2):
        return pl.BlockSpec((1, d1, d2), lambda c, l, ids: (l, 0, 0))

    grid_spec = pltpu.PrefetchScalarGridSpec(
        num_scalar_prefetch=1,
        grid=(_NCORES, L),
        in_specs=[
            pl.BlockSpec(memory_space=pl.ANY),
            cmap((S, H)),
            cmap((1, H)),
            cmap((1, H)), cmap((1, H)),
            pl.BlockSpec((1, R, 1, S), lambda c, l, ids: (c, 0, 0, 0)),
            wspec(H, 3 * H), wspec(1, 3 * H),
            wspec(H, H), wspec(1, H),
            wspec(1, H), wspec(1, H),
            wspec(H, I), wspec(1, I),
            wspec(I, H), wspec(1, H),
            wspec(1, H), wspec(1, H),
        ],
        out_specs=pl.BlockSpec((1, T, H), lambda c, l, ids: (c, 0, 0)),
        scratch_shapes=[
            pltpu.VMEM((T, 3 * H), jnp.float32),
            pltpu.VMEM((R * nh // 2, S, 2 * hd), jnp.bfloat16),
            pltpu.VMEM((R * nh // 2, 2 * S, 2 * hd), jnp.bfloat16),
            pltpu.VMEM((R * nh // 2, S, 2 * S), jnp.bfloat16),
            pltpu.VMEM((R * nh // 2, 2 * S, 2 * hd), jnp.bfloat16),
            pltpu.VMEM((T, H), jnp.bfloat16),
            pltpu.SemaphoreType.DMA((16,)),
        ],
    )

    out = pl.pallas_call(
        functools.partial(_fused_kernel, S),
        out_shape=jax.ShapeDtypeStruct((_NCORES, T, H), jnp.float32),
        grid_spec=grid_spec,
        compiler_params=pltpu.CompilerParams(
            dimension_semantics=("core_parallel", "arbitrary"),
            vmem_limit_bytes=56 * 1024 * 1024),
    )(ids2, word_emb, pos_emb[:S], type_emb[0:1],
      emb_ln_g.reshape(1, H), emb_ln_b.reshape(1, H), mask_bias,
      wqkv, bqkv, wo, bo, ln1_g, ln1_b, w1, b1, w2, b2, ln2_g, ln2_b)

    return out.reshape(B, S, H)


# speedup vs baseline: 1.4191x; 1.0043x over previous
import functools
import math

import jax
import jax.numpy as jnp
from jax.experimental import pallas as pl
from jax.experimental.pallas import tpu as pltpu

_LN_EPS = 1e-12
_INV_SQRT2 = 0.7071067811865476
_NUM_HEADS = 8
_NCORES = 1


def _ln(x, g, b):
    mu = jnp.mean(x, axis=-1, keepdims=True)
    ex2 = jnp.mean(x * x, axis=-1, keepdims=True)
    var = jnp.maximum(ex2 - mu * mu, 0.0)
    return (x - mu) * jax.lax.rsqrt(var + _LN_EPS) * g + b


def _gelu(x):
    return 0.5 * x * (1.0 + jax.lax.erf(x * _INV_SQRT2))


def _fused_kernel(S,
                  ids_ref,
                  wemb_hbm,
                  posx_ref, typ_ref, embg_ref, embb_ref,
                  mask_ref,
                  wqkv_ref, bqkv_ref, wo_ref, bo_ref,
                  l1g_ref, l1b_ref, w1_ref, b1_ref, w2_ref, b2_ref,
                  l2g_ref, l2b_ref,
                  o_ref,
                  qkv_scr, qpair_scr, kbd_scr, pair_scr, vbd_scr,
                  ctx_scr, sems):
    c = pl.program_id(0)
    l = pl.program_id(1)
    n_layers = pl.num_programs(1)

    T, H = o_ref.shape[1], o_ref.shape[2]
    nh = _NUM_HEADS
    hd = H // nh
    R = T // S
    nb = R * nh
    scale = 1.0 / math.sqrt(hd)
    bf16, f32 = jnp.bfloat16, jnp.float32

    NSEM = 16
    BANK = T // NSEM

    @pl.when(l == 0)
    def _():
        def issue(i, carry):
            for j in range(8):
                tid = ids_ref[c, i * 8 + j]
                pltpu.make_async_copy(
                    wemb_hbm.at[pl.ds(tid, 1)],
                    o_ref.at[0, pl.ds(i * 8 + j, 1)],
                    sems.at[jnp.bitwise_and(i * 8 + j, NSEM - 1)]).start()
            return carry
        jax.lax.fori_loop(0, T // 8, issue, 0)

        for s in range(NSEM):
            pltpu.make_async_copy(
                wemb_hbm.at[pl.ds(0, BANK)],
                o_ref.at[0, pl.ds(0, BANK)],
                sems.at[s]).wait()

        S_, H_ = posx_ref.shape
        posT = posx_ref[...] + typ_ref[...]
        emb = o_ref[0].reshape(T // S_, S_, H_) + posT[None]
        o_ref[0] = _ln(emb, embg_ref[...], embb_ref[...]).reshape(T, H)

    @pl.when(l == 0)
    def _():
        vbd_scr[...] = jnp.zeros_like(vbd_scr)
        kbd_scr[...] = jnp.zeros_like(kbd_scr)

    x = o_ref[0]

    qkv_scr[...] = jnp.dot(x.astype(bf16), wqkv_ref[0],
                           preferred_element_type=f32)
    bq = bqkv_ref[0]
    for r in range(R):
        rs = pl.ds(r * S, S)
        for hp in range(nh // 2):
            b2 = r * (nh // 2) + hp
            qpair_scr[b2] = ((qkv_scr[rs, pl.ds(2 * hp * hd, 2 * hd)]
                              + bq[:, 2 * hp * hd:(2 * hp + 2) * hd])
                             * scale).astype(bf16)
        for h in range(nh):
            b2 = (r * nh + h) // 2
            sl = (h % 2) * hd
            kbd_scr[b2, pl.ds((h % 2) * S, S), pl.ds(sl, hd)] = (
                qkv_scr[rs, pl.ds(H + h * hd, hd)]
                + bq[:, H + h * hd:H + (h + 1) * hd]).astype(bf16)
            vbd_scr[b2, pl.ds((h % 2) * S, S), pl.ds(sl, hd)] = (
                qkv_scr[rs, pl.ds(2 * H + h * hd, hd)]
                + bq[:, 2 * H + h * hd:2 * H + (h + 1) * hd]).astype(bf16)

    sc = jnp.einsum("bqd,bkd->bqk", qpair_scr[...], kbd_scr[...],
                    preferred_element_type=f32)
    msk = mask_ref[0]
    m2 = jnp.concatenate([msk, msk], axis=-1)
    sc = sc + jnp.broadcast_to(m2[:, None], (R, nh // 2, 1, 2 * S)).reshape(
        nb // 2, 1, 2 * S)
    p = jnp.exp(sc)
    r0 = pl.reciprocal(jnp.sum(p[:, :, :S], axis=-1, keepdims=True), approx=True)
    r1 = pl.reciprocal(jnp.sum(p[:, :, S:], axis=-1, keepdims=True), approx=True)
    pair_scr[:, :, pl.ds(0, S)] = (p[:, :, :S] * r0).astype(bf16)
    pair_scr[:, :, pl.ds(S, S)] = (p[:, :, S:] * r1).astype(bf16)
    ctx = jnp.einsum("bqk,bkd->bqd", pair_scr[...], vbd_scr[...],
                     preferred_element_type=f32)

    for r in range(R):
        rs = pl.ds(r * S, S)
        for hp in range(nh // 2):
            ctx_scr[rs, pl.ds(hp * 2 * hd, 2 * hd)] = ctx[r * nh // 2 + hp].astype(bf16)

    attn = jnp.dot(ctx_scr[...], wo_ref[0],
                   preferred_element_type=f32) + bo_ref[0]
    h1 = _ln(x + attn, l1g_ref[0], l1b_ref[0])

    ff = jnp.dot(h1.astype(bf16), w1_ref[0],
                 preferred_element_type=f32) + b1_ref[0]
    ff = _gelu(ff)
    ff2 = jnp.dot(ff.astype(bf16), w2_ref[0],
                  preferred_element_type=f32) + b2_ref[0]

    o_ref[0] = _ln(h1 + ff2, l2g_ref[0], l2b_ref[0])


def kernel(word_emb, pos_emb, type_emb, emb_ln_g, emb_ln_b,
           wqkv, bqkv, wo, bo, ln1_g, ln1_b, w1, b1, w2, b2,
           ln2_g, ln2_b, input_ids, attention_mask):
    B, S = input_ids.shape
    H = word_emb.shape[1]
    L = wqkv.shape[0]
    I = w1.shape[2]
    nh = _NUM_HEADS
    hd = H // nh
    R = B // _NCORES
    T = R * S

    if attention_mask is None:
        attention_mask = jnp.ones((B, S), jnp.float32)
    mask_bias = ((1.0 - attention_mask.astype(jnp.float32))
                 * jnp.finfo(jnp.float32).min).reshape(_NCORES, R, 1, S)
    ids2 = input_ids.astype(jnp.int32).reshape(_NCORES, T)

    def cmap(shape):
        return pl.BlockSpec(shape, lambda c, l, ids: (0,) * len(shape))

    def wspec(d1, d2):
        return pl.BlockSpec((1, d1, d2), lambda c, l, ids: (l, 0, 0))

    grid_spec = pltpu.PrefetchScalarGridSpec(
        num_scalar_prefetch=1,
        grid=(_NCORES, L),
        in_specs=[
            pl.BlockSpec(memory_space=pl.ANY),
            cmap((S, H)),
            cmap((1, H)),
            cmap((1, H)), cmap((1, H)),
            pl.BlockSpec((1, R, 1, S), lambda c, l, ids: (c, 0, 0, 0)),
            wspec(H, 3 * H), wspec(1, 3 * H),
            wspec(H, H), wspec(1, H),
            wspec(1, H), wspec(1, H),
            wspec(H, I), wspec(1, I),
            wspec(I, H), wspec(1, H),
            wspec(1, H), wspec(1, H),
        ],
        out_specs=pl.BlockSpec((1, T, H), lambda c, l, ids: (c, 0, 0)),
        scratch_shapes=[
            pltpu.VMEM((T, 3 * H), jnp.float32),
            pltpu.VMEM((R * nh // 2, S, 2 * hd), jnp.bfloat16),
            pltpu.VMEM((R * nh // 2, 2 * S, 2 * hd), jnp.bfloat16),
            pltpu.VMEM((R * nh // 2, S, 2 * S), jnp.bfloat16),
            pltpu.VMEM((R * nh // 2, 2 * S, 2 * hd), jnp.bfloat16),
            pltpu.VMEM((T, H), jnp.bfloat16),
            pltpu.SemaphoreType.DMA((16,)),
        ],
    )

    out = pl.pallas_call(
        functools.partial(_fused_kernel, S),
        out_shape=jax.ShapeDtypeStruct((_NCORES, T, H), jnp.float32),
        grid_spec=grid_spec,
        compiler_params=pltpu.CompilerParams(
            dimension_semantics=("core_parallel", "arbitrary"),
            vmem_limit_bytes=56 * 1024 * 1024),
    )(ids2, word_emb, pos_emb[:S], type_emb[0:1],
      emb_ln_g.reshape(1, H), emb_ln_b.reshape(1, H), mask_bias,
      wqkv, bqkv, wo, bo, ln1_g, ln1_b, w1, b1, w2, b2, ln2_g, ln2_b)

    return out.reshape(B, S, H)


# speedup vs baseline: 1.4326x; 1.0095x over previous
import functools
import math

import jax
import jax.numpy as jnp
from jax.experimental import pallas as pl
from jax.experimental.pallas import tpu as pltpu

_LN_EPS = 1e-12
_INV_SQRT2 = 0.7071067811865476
_NUM_HEADS = 8
_NCORES = 1


def _ln(x, g, b):
    mu = jnp.mean(x, axis=-1, keepdims=True)
    ex2 = jnp.mean(x * x, axis=-1, keepdims=True)
    var = jnp.maximum(ex2 - mu * mu, 0.0)
    return (x - mu) * jax.lax.rsqrt(var + _LN_EPS) * g + b


def _gelu(x):
    return 0.5 * x * (1.0 + jax.lax.erf(x * _INV_SQRT2))


def _fused_kernel(S,
                  ids_ref,
                  wemb_hbm,
                  posx_ref, typ_ref, embg_ref, embb_ref,
                  mask_ref,
                  wqkv_ref, bqkv_ref, wo_ref, bo_ref,
                  l1g_ref, l1b_ref, w1_ref, b1_ref, w2_ref, b2_ref,
                  l2g_ref, l2b_ref,
                  o_ref,
                  qkv_scr, qpair_scr, kbd_scr, pair_scr, vbd_scr,
                  ctx_scr, sems):
    c = pl.program_id(0)
    l = pl.program_id(1)
    n_layers = pl.num_programs(1)

    T, H = o_ref.shape[1], o_ref.shape[2]
    nh = _NUM_HEADS
    hd = H // nh
    R = T // S
    nb = R * nh
    scale = 1.0 / math.sqrt(hd)
    bf16, f32 = jnp.bfloat16, jnp.float32

    NSEM = 8
    BANK = T // NSEM

    @pl.when(l == 0)
    def _():
        def issue(i, carry):
            base = i * NSEM
            for j in range(NSEM):
                tid = ids_ref[c, base + j]
                pltpu.make_async_copy(
                    wemb_hbm.at[pl.ds(tid, 1)],
                    o_ref.at[0, pl.ds(base + j, 1)],
                    sems.at[j]).start()
            return carry
        jax.lax.fori_loop(0, T // NSEM, issue, 0, unroll=2)

        for s in range(NSEM):
            pltpu.make_async_copy(
                wemb_hbm.at[pl.ds(0, BANK)],
                o_ref.at[0, pl.ds(0, BANK)],
                sems.at[s]).wait()

        S_, H_ = posx_ref.shape
        posT = posx_ref[...] + typ_ref[...]
        emb = o_ref[0].reshape(T // S_, S_, H_) + posT[None]
        o_ref[0] = _ln(emb, embg_ref[...], embb_ref[...]).reshape(T, H)

    @pl.when(l == 0)
    def _():
        vbd_scr[...] = jnp.zeros_like(vbd_scr)
        kbd_scr[...] = jnp.zeros_like(kbd_scr)

    x = o_ref[0]

    qkv_scr[...] = jnp.dot(x.astype(bf16), wqkv_ref[0],
                           preferred_element_type=f32)
    bq = bqkv_ref[0]
    for r in range(R):
        rs = pl.ds(r * S, S)
        for hp in range(nh // 2):
            b2 = r * (nh // 2) + hp
            qpair_scr[b2] = ((qkv_scr[rs, pl.ds(2 * hp * hd, 2 * hd)]
                              + bq[:, 2 * hp * hd:(2 * hp + 2) * hd])
                             * scale).astype(bf16)
        for h in range(nh):
            b2 = (r * nh + h) // 2
            sl = (h % 2) * hd
            kbd_scr[b2, pl.ds((h % 2) * S, S), pl.ds(sl, hd)] = (
                qkv_scr[rs, pl.ds(H + h * hd, hd)]
                + bq[:, H + h * hd:H + (h + 1) * hd]).astype(bf16)
            vbd_scr[b2, pl.ds((h % 2) * S, S), pl.ds(sl, hd)] = (
                qkv_scr[rs, pl.ds(2 * H + h * hd, hd)]
                + bq[:, 2 * H + h * hd:2 * H + (h + 1) * hd]).astype(bf16)

    sc = jnp.einsum("bqd,bkd->bqk", qpair_scr[...], kbd_scr[...],
                    preferred_element_type=f32)
    msk = mask_ref[0]
    m2 = jnp.concatenate([msk, msk], axis=-1)
    sc = sc + jnp.broadcast_to(m2[:, None], (R, nh // 2, 1, 2 * S)).reshape(
        nb // 2, 1, 2 * S)
    p = jnp.exp(sc)
    r0 = pl.reciprocal(jnp.sum(p[:, :, :S], axis=-1, keepdims=True), approx=True)
    r1 = pl.reciprocal(jnp.sum(p[:, :, S:], axis=-1, keepdims=True), approx=True)
    pair_scr[:, :, pl.ds(0, S)] = (p[:, :, :S] * r0).astype(bf16)
    pair_scr[:, :, pl.ds(S, S)] = (p[:, :, S:] * r1).astype(bf16)
    ctx = jnp.einsum("bqk,bkd->bqd", pair_scr[...], vbd_scr[...],
                     preferred_element_type=f32)

    for r in range(R):
        rs = pl.ds(r * S, S)
        for hp in range(nh // 2):
            ctx_scr[rs, pl.ds(hp * 2 * hd, 2 * hd)] = ctx[r * nh // 2 + hp].astype(bf16)

    attn = jnp.dot(ctx_scr[...], wo_ref[0],
                   preferred_element_type=f32) + bo_ref[0]
    h1 = _ln(x + attn, l1g_ref[0], l1b_ref[0])

    ff = jnp.dot(h1.astype(bf16), w1_ref[0],
                 preferred_element_type=f32) + b1_ref[0]
    ff = _gelu(ff)
    ff2 = jnp.dot(ff.astype(bf16), w2_ref[0],
                  preferred_element_type=f32) + b2_ref[0]

    o_ref[0] = _ln(h1 + ff2, l2g_ref[0], l2b_ref[0])


def kernel(word_emb, pos_emb, type_emb, emb_ln_g, emb_ln_b,
           wqkv, bqkv, wo, bo, ln1_g, ln1_b, w1, b1, w2, b2,
           ln2_g, ln2_b, input_ids, attention_mask):
    B, S = input_ids.shape
    H = word_emb.shape[1]
    L = wqkv.shape[0]
    I = w1.shape[2]
    nh = _NUM_HEADS
    hd = H // nh
    R = B // _NCORES
    T = R * S

    if attention_mask is None:
        attention_mask = jnp.ones((B, S), jnp.float32)
    mask_bias = ((1.0 - attention_mask.astype(jnp.float32))
                 * jnp.finfo(jnp.float32).min).reshape(_NCORES, R, 1, S)
    ids2 = input_ids.astype(jnp.int32).reshape(_NCORES, T)

    def cmap(shape):
        return pl.BlockSpec(shape, lambda c, l, ids: (0,) * len(shape))

    def wspec(d1, d2):
        return pl.BlockSpec((1, d1, d2), lambda c, l, ids: (l, 0, 0))

    grid_spec = pltpu.PrefetchScalarGridSpec(
        num_scalar_prefetch=1,
        grid=(_NCORES, L),
        in_specs=[
            pl.BlockSpec(memory_space=pl.ANY),
            cmap((S, H)),
            cmap((1, H)),
            cmap((1, H)), cmap((1, H)),
            pl.BlockSpec((1, R, 1, S), lambda c, l, ids: (c, 0, 0, 0)),
            wspec(H, 3 * H), wspec(1, 3 * H),
            wspec(H, H), wspec(1, H),
            wspec(1, H), wspec(1, H),
            wspec(H, I), wspec(1, I),
            wspec(I, H), wspec(1, H),
            wspec(1, H), wspec(1, H),
        ],
        out_specs=pl.BlockSpec((1, T, H), lambda c, l, ids: (c, 0, 0)),
        scratch_shapes=[
            pltpu.VMEM((T, 3 * H), jnp.float32),
            pltpu.VMEM((R * nh // 2, S, 2 * hd), jnp.bfloat16),
            pltpu.VMEM((R * nh // 2, 2 * S, 2 * hd), jnp.bfloat16),
            pltpu.VMEM((R * nh // 2, S, 2 * S), jnp.bfloat16),
            pltpu.VMEM((R * nh // 2, 2 * S, 2 * hd), jnp.bfloat16),
            pltpu.VMEM((T, H), jnp.bfloat16),
            pltpu.SemaphoreType.DMA((16,)),
        ],
    )

    out = pl.pallas_call(
        functools.partial(_fused_kernel, S),
        out_shape=jax.ShapeDtypeStruct((_NCORES, T, H), jnp.float32),
        grid_spec=grid_spec,
        compiler_params=pltpu.CompilerParams(
            dimension_semantics=("core_parallel", "arbitrary"),
            vmem_limit_bytes=56 * 1024 * 1024),
    )(ids2, word_emb, pos_emb[:S], type_emb[0:1],
      emb_ln_g.reshape(1, H), emb_ln_b.reshape(1, H), mask_bias,
      wqkv, bqkv, wo, bo, ln1_g, ln1_b, w1, b1, w2, b2, ln2_g, ln2_b)

    return out.reshape(B, S, H)


# speedup vs baseline: 1.4413x; 1.0061x over previous
import functools
import math

import jax
import jax.numpy as jnp
from jax.experimental import pallas as pl
from jax.experimental.pallas import tpu as pltpu

_LN_EPS = 1e-12
_INV_SQRT2 = 0.7071067811865476
_NUM_HEADS = 8
_NCORES = 1


def _ln(x, g, b):
    mu = jnp.mean(x, axis=-1, keepdims=True)
    ex2 = jnp.mean(x * x, axis=-1, keepdims=True)
    var = jnp.maximum(ex2 - mu * mu, 0.0)
    return (x - mu) * jax.lax.rsqrt(var + _LN_EPS) * g + b


def _gelu(x):
    return 0.5 * x * (1.0 + jax.lax.erf(x * _INV_SQRT2))


def _fused_kernel(S,
                  ids_ref,
                  wemb_hbm,
                  posx_ref, typ_ref, embg_ref, embb_ref,
                  mask_ref,
                  wqkv_ref, bqkv_ref, wo_ref, bo_ref,
                  l1g_ref, l1b_ref, w1_ref, b1_ref, w2_ref, b2_ref,
                  l2g_ref, l2b_ref,
                  o_ref,
                  qkv_scr, qpair_scr, kbd_scr, pair_scr, vbd_scr,
                  ctx_scr, sems):
    c = pl.program_id(0)
    l = pl.program_id(1)
    n_layers = pl.num_programs(1)

    T, H = o_ref.shape[1], o_ref.shape[2]
    nh = _NUM_HEADS
    hd = H // nh
    R = T // S
    nb = R * nh
    scale = 1.0 / math.sqrt(hd)
    bf16, f32 = jnp.bfloat16, jnp.float32

    NSEM = 8
    BANK = T // NSEM

    @pl.when(l == 0)
    def _():
        def issue(i, carry):
            base = i * NSEM
            for j in range(NSEM):
                tid = ids_ref[c, base + j]
                pltpu.make_async_copy(
                    wemb_hbm.at[pl.ds(tid, 1)],
                    o_ref.at[0, pl.ds(base + j, 1)],
                    sems.at[j]).start()
            return carry
        jax.lax.fori_loop(0, T // NSEM, issue, 0, unroll=2)

        for s in range(NSEM):
            pltpu.make_async_copy(
                wemb_hbm.at[pl.ds(0, BANK)],
                o_ref.at[0, pl.ds(0, BANK)],
                sems.at[s]).wait()

        S_, H_ = posx_ref.shape
        posT = posx_ref[...] + typ_ref[...]
        emb = o_ref[0].reshape(T // S_, S_, H_) + posT[None]
        o_ref[0] = _ln(emb, embg_ref[...], embb_ref[...]).reshape(T, H)

    @pl.when(l == 0)
    def _():
        vbd_scr[...] = jnp.zeros_like(vbd_scr)
        kbd_scr[...] = jnp.zeros_like(kbd_scr)

    x = o_ref[0]

    qkv_scr[...] = jnp.dot(x.astype(bf16), wqkv_ref[0],
                           preferred_element_type=f32)
    bq = bqkv_ref[0]
    for r in range(R):
        rs = pl.ds(r * S, S)
        for hp in range(nh // 2):
            b2 = r * (nh // 2) + hp
            qpair_scr[b2] = ((qkv_scr[rs, pl.ds(2 * hp * hd, 2 * hd)]
                              + bq[:, 2 * hp * hd:(2 * hp + 2) * hd])
                             * scale).astype(bf16)
        for h in range(nh):
            b2 = (r * nh + h) // 2
            sl = (h % 2) * hd
            kbd_scr[b2, pl.ds((h % 2) * S, S), pl.ds(sl, hd)] = (
                qkv_scr[rs, pl.ds(H + h * hd, hd)]
                + bq[:, H + h * hd:H + (h + 1) * hd]).astype(bf16)
            vbd_scr[b2, pl.ds((h % 2) * S, S), pl.ds(sl, hd)] = (
                qkv_scr[rs, pl.ds(2 * H + h * hd, hd)]
                + bq[:, 2 * H + h * hd:2 * H + (h + 1) * hd]).astype(bf16)

    sc = jnp.einsum("bqd,bkd->bqk", qpair_scr[...], kbd_scr[...],
                    preferred_element_type=f32)
    msk = mask_ref[0]
    m2 = jnp.concatenate([msk, msk], axis=-1)
    sc = sc + jnp.broadcast_to(m2[:, None], (R, nh // 2, 1, 2 * S)).reshape(
        nb // 2, 1, 2 * S)
    p = jnp.exp(sc)
    r0 = pl.reciprocal(jnp.sum(p[:, :, :S], axis=-1, keepdims=True), approx=True)
    r1 = pl.reciprocal(jnp.sum(p[:, :, S:], axis=-1, keepdims=True), approx=True)
    pair_scr[:, :, pl.ds(0, S)] = (p[:, :, :S] * r0).astype(bf16)
    pair_scr[:, :, pl.ds(S, S)] = (p[:, :, S:] * r1).astype(bf16)
    ctx = jnp.einsum("bqk,bkd->bqd", pair_scr[...], vbd_scr[...],
                     preferred_element_type=f32)

    for r in range(R):
        rs = pl.ds(r * S, S)
        for hp in range(nh // 2):
            ctx_scr[rs, pl.ds(hp * 2 * hd, 2 * hd)] = ctx[r * nh // 2 + hp].astype(bf16)

    attn = jnp.dot(ctx_scr[...], wo_ref[0],
                   preferred_element_type=f32) + bo_ref[0]
    h1 = _ln(x + attn, l1g_ref[0], l1b_ref[0])

    Ihalf = w1_ref.shape[2] // 2
    h1b = h1.astype(bf16)
    ff2 = b2_ref[0]
    for ih in range(2):
        isl = slice(ih * Ihalf, (ih + 1) * Ihalf)
        ffh = jnp.dot(h1b, w1_ref[0][:, isl],
                      preferred_element_type=f32) + b1_ref[0][:, isl]
        ffh = _gelu(ffh)
        ff2 = ff2 + jnp.dot(ffh.astype(bf16), w2_ref[0][isl],
                            preferred_element_type=f32)

    o_ref[0] = _ln(h1 + ff2, l2g_ref[0], l2b_ref[0])


def kernel(word_emb, pos_emb, type_emb, emb_ln_g, emb_ln_b,
           wqkv, bqkv, wo, bo, ln1_g, ln1_b, w1, b1, w2, b2,
           ln2_g, ln2_b, input_ids, attention_mask):
    B, S = input_ids.shape
    H = word_emb.shape[1]
    L = wqkv.shape[0]
    I = w1.shape[2]
    nh = _NUM_HEADS
    hd = H // nh
    R = B // _NCORES
    T = R * S

    if attention_mask is None:
        attention_mask = jnp.ones((B, S), jnp.float32)
    mask_bias = ((1.0 - attention_mask.astype(jnp.float32))
                 * jnp.finfo(jnp.float32).min).reshape(_NCORES, R, 1, S)
    ids2 = input_ids.astype(jnp.int32).reshape(_NCORES, T)

    def cmap(shape):
        return pl.BlockSpec(shape, lambda c, l, ids: (0,) * len(shape))

    def wspec(d1, d2):
        return pl.BlockSpec((1, d1, d2), lambda c, l, ids: (l, 0, 0))

    grid_spec = pltpu.PrefetchScalarGridSpec(
        num_scalar_prefetch=1,
        grid=(_NCORES, L),
        in_specs=[
            pl.BlockSpec(memory_space=pl.ANY),
            cmap((S, H)),
            cmap((1, H)),
            cmap((1, H)), cmap((1, H)),
            pl.BlockSpec((1, R, 1, S), lambda c, l, ids: (c, 0, 0, 0)),
            wspec(H, 3 * H), wspec(1, 3 * H),
            wspec(H, H), wspec(1, H),
            wspec(1, H), wspec(1, H),
            wspec(H, I), wspec(1, I),
            wspec(I, H), wspec(1, H),
            wspec(1, H), wspec(1, H),
        ],
        out_specs=pl.BlockSpec((1, T, H), lambda c, l, ids: (c, 0, 0)),
        scratch_shapes=[
            pltpu.VMEM((T, 3 * H), jnp.float32),
            pltpu.VMEM((R * nh // 2, S, 2 * hd), jnp.bfloat16),
            pltpu.VMEM((R * nh // 2, 2 * S, 2 * hd), jnp.bfloat16),
            pltpu.VMEM((R * nh // 2, S, 2 * S), jnp.bfloat16),
            pltpu.VMEM((R * nh // 2, 2 * S, 2 * hd), jnp.bfloat16),
            pltpu.VMEM((T, H), jnp.bfloat16),
            pltpu.SemaphoreType.DMA((16,)),
        ],
    )

    out = pl.pallas_call(
        functools.partial(_fused_kernel, S),
        out_shape=jax.ShapeDtypeStruct((_NCORES, T, H), jnp.float32),
        grid_spec=grid_spec,
        compiler_params=pltpu.CompilerParams(
            dimension_semantics=("core_parallel", "arbitrary"),
            vmem_limit_bytes=56 * 1024 * 1024),
    )(ids2, word_emb, pos_emb[:S], type_emb[0:1],
      emb_ln_g.reshape(1, H), emb_ln_b.reshape(1, H), mask_bias,
      wqkv, bqkv, wo, bo, ln1_g, ln1_b, w1, b1, w2, b2, ln2_g, ln2_b)

    return out.reshape(B, S, H)
